# Initial kernel scaffold; baseline (speedup 1.0000x reference)
import functools

import jax
import jax.numpy as jnp
from jax import lax
from jax.experimental import pallas as pl
from jax.experimental.pallas import tpu as pltpu
from jax.experimental.pallas import tpu_sc as plsc

N = 10000
E = 320000
D = 128

CHUNK = 128
NCHUNKS = E // CHUNK
NC, NS, L = 2, 16, 16
NW = NC * NS
NP = 10240
CHUNKS_PER_CORE = NCHUNKS // NC
ITERS = -(-CHUNKS_PER_CORE // NS)

_mesh = plsc.VectorSubcoreMesh(core_axis_name="c", subcore_axis_name="s")


@functools.partial(
    pl.kernel,
    out_type=jax.ShapeDtypeStruct((NC, NP, L), jnp.float32),
    mesh=_mesh,
    scratch_types=[
        pltpu.VMEM((1, CHUNK), jnp.int32),
        pltpu.VMEM((CHUNK, L), jnp.float32),
        pltpu.VMEM_SHARED((NP, L), jnp.float32),
    ],
)
def _deg_kernel(dst_hbm, out_hbm, idx_v, ones_v, hist_sh):
    cid = lax.axis_index("c")
    sid = lax.axis_index("s")

    one16 = jnp.ones((L,), jnp.float32)
    zero16 = jnp.zeros((L,), jnp.float32)
    for r in range(32):
        ones_v[r, :] = zero16
    rows_per_sub = NP // NS
    def _zero_body(i, carry):
        pltpu.sync_copy(
            ones_v.at[pl.ds(0, 32)],
            hist_sh.at[pl.ds(sid * rows_per_sub + i * 32, 32)],
        )
        return carry
    lax.fori_loop(0, rows_per_sub // 32, _zero_body, 0)
    for r in range(CHUNK):
        ones_v[r, :] = one16
    plsc.subcore_barrier()

    def _body(i, carry):
        rel = sid + NS * i
        @pl.when(rel < CHUNKS_PER_CORE)
        def _():
            k = cid * CHUNKS_PER_CORE + rel
            pltpu.sync_copy(dst_hbm.at[k], idx_v.at[0])
            pltpu.sync_copy(ones_v, hist_sh.at[idx_v.at[0]], add=True)
        return carry
    lax.fori_loop(0, ITERS, _body, 0)
    plsc.subcore_barrier()

    pltpu.sync_copy(
        hist_sh.at[pl.ds(sid * rows_per_sub, rows_per_sub)],
        out_hbm.at[cid, pl.ds(sid * rows_per_sub, rows_per_sub)],
    )


@functools.partial(
    pl.kernel,
    out_type=jax.ShapeDtypeStruct((NC, NP, D), jnp.float32),
    mesh=_mesh,
    scratch_types=[
        pltpu.VMEM((2, CHUNK), jnp.int32),
        pltpu.VMEM((CHUNK, D), jnp.float32),
        pltpu.VMEM((32, D), jnp.float32),
        pltpu.VMEM_SHARED((NP, D), jnp.float32),
    ],
)
def _agg_kernel(xws_hbm, src_hbm, dst_hbm, out_hbm, idx_v, rows_v, z_v, acc_sh):
    cid = lax.axis_index("c")
    sid = lax.axis_index("s")

    zero16 = jnp.zeros((L,), jnp.float32)
    for r in range(32):
        for j in range(D // L):
            z_v[r, pl.ds(j * L, L)] = zero16
    rows_per_sub = NP // NS
    def _zero_body(i, carry):
        pltpu.sync_copy(z_v, acc_sh.at[pl.ds(sid * rows_per_sub + i * 32, 32)])
        return carry
    lax.fori_loop(0, rows_per_sub // 32, _zero_body, 0)
    plsc.subcore_barrier()

    def _body(i, carry):
        rel = sid + NS * i
        @pl.when(rel < CHUNKS_PER_CORE)
        def _():
            k = cid * CHUNKS_PER_CORE + rel
            pltpu.sync_copy(src_hbm.at[k], idx_v.at[0])
            pltpu.sync_copy(dst_hbm.at[k], idx_v.at[1])
            pltpu.sync_copy(xws_hbm.at[idx_v.at[0]], rows_v)
            pltpu.sync_copy(rows_v, acc_sh.at[idx_v.at[1]], add=True)
        return carry
    lax.fori_loop(0, ITERS, _body, 0)
    plsc.subcore_barrier()

    pltpu.sync_copy(
        acc_sh.at[pl.ds(sid * rows_per_sub, rows_per_sub)],
        out_hbm.at[cid, pl.ds(sid * rows_per_sub, rows_per_sub)],
    )


def _mm_body(x_ref, w_ref, dp_ref, xws_ref):
    dp = dp_ref[...]
    deg = dp[0, :, 0] + dp[1, :, 0] + 1.0
    dis = lax.rsqrt(deg)
    xw = jax.lax.dot_general(
        x_ref[...], w_ref[...], (((1,), (0,)), ((), ())),
        preferred_element_type=jnp.float32,
        precision=jax.lax.Precision.HIGHEST,
    )
    xws_ref[...] = xw * dis[:, None]


BM = 2000


def _mm(x, W, dp):
    return pl.pallas_call(
        _mm_body,
        grid=(N // BM,),
        in_specs=[
            pl.BlockSpec((BM, D), lambda i: (i, 0)),
            pl.BlockSpec((D, D), lambda i: (0, 0)),
            pl.BlockSpec((NC, BM, L), lambda i: (0, i, 0)),
        ],
        out_specs=pl.BlockSpec((BM, D), lambda i: (i, 0)),
        out_shape=jax.ShapeDtypeStruct((N, D), jnp.float32),
    )(x, W, dp)


def _comb_body(p_ref, xws_ref, dp_ref, b_ref, o_ref):
    dp = dp_ref[...]
    deg = dp[0, :, 0] + dp[1, :, 0] + 1.0
    dis = lax.rsqrt(deg)
    p = p_ref[...]
    o_ref[...] = dis[:, None] * (p[0] + p[1] + xws_ref[...]) + b_ref[...][None, :]


def _combine(p, xws, dp, b):
    return pl.pallas_call(
        _comb_body,
        grid=(N // BM,),
        in_specs=[
            pl.BlockSpec((NC, BM, D), lambda i: (0, i, 0)),
            pl.BlockSpec((BM, D), lambda i: (i, 0)),
            pl.BlockSpec((NC, BM, L), lambda i: (0, i, 0)),
            pl.BlockSpec((D,), lambda i: (0,)),
        ],
        out_specs=pl.BlockSpec((BM, D), lambda i: (i, 0)),
        out_shape=jax.ShapeDtypeStruct((N, D), jnp.float32),
    )(p, xws, dp, b)


def kernel(x, edge_index, W, b):
    src2d = edge_index[0].reshape(NCHUNKS, CHUNK)
    dst2d = edge_index[1].reshape(NCHUNKS, CHUNK)
    dp = _deg_kernel(dst2d)
    xws = _mm(x, W, dp)
    p = _agg_kernel(xws, src2d, dst2d)
    return _combine(p, xws, dp, b)


# speedup vs baseline: 21.4229x; 21.4229x over previous
import functools

import jax
import jax.numpy as jnp
from jax import lax
from jax.experimental import pallas as pl
from jax.experimental.pallas import tpu as pltpu
from jax.experimental.pallas import tpu_sc as plsc

N = 10000
E = 320000
D = 128

CHUNK = 128
NCHUNKS = E // CHUNK
NC, NS, L = 2, 16, 16
NW = NC * NS
NP = 10240
CHUNKS_PER_CORE = NCHUNKS // NC
ITERS = -(-CHUNKS_PER_CORE // NS)

_mesh = plsc.VectorSubcoreMesh(core_axis_name="c", subcore_axis_name="s")


def _deg_kernel_body(dst_hbm, out_hbm, idx_v, ones_v, hist_sh):
    cid = lax.axis_index("c")
    sid = lax.axis_index("s")

    one16 = jnp.ones((L,), jnp.float32)
    zero16 = jnp.zeros((L,), jnp.float32)
    for r in range(32):
        ones_v[r, :] = zero16
    rows_per_sub = NP // NS
    def _zero_body(i, carry):
        pltpu.sync_copy(
            ones_v.at[pl.ds(0, 32)],
            hist_sh.at[pl.ds(sid * rows_per_sub + i * 32, 32)],
        )
        return carry
    lax.fori_loop(0, rows_per_sub // 32, _zero_body, 0)
    for r in range(CHUNK):
        ones_v[r, :] = one16
    plsc.subcore_barrier()

    def _body(i, carry):
        rel = sid + NS * i
        @pl.when(rel < CHUNKS_PER_CORE)
        def _():
            k = cid * CHUNKS_PER_CORE + rel
            pltpu.sync_copy(dst_hbm.at[k], idx_v.at[0])
            pltpu.sync_copy(ones_v, hist_sh.at[idx_v.at[0]], add=True)
        return carry
    lax.fori_loop(0, ITERS, _body, 0)
    plsc.subcore_barrier()

    pltpu.sync_copy(
        hist_sh.at[pl.ds(sid * rows_per_sub, rows_per_sub)],
        out_hbm.at[cid, pl.ds(sid * rows_per_sub, rows_per_sub)],
    )


_deg_kernel = functools.partial(
    pl.kernel,
    out_type=jax.ShapeDtypeStruct((NC, NP, L), jnp.float32),
    mesh=_mesh,
    scratch_types=[
        pltpu.VMEM((1, CHUNK), jnp.int32),
        pltpu.VMEM((CHUNK, L), jnp.float32),
        pltpu.VMEM_SHARED((NP, L), jnp.float32),
    ],
)(_deg_kernel_body)


def _agg_kernel_body(xws_hbm, src_hbm, dst_hbm, out_hbm, idx_v, rows_v, z_v, acc_sh):
    cid = lax.axis_index("c")
    sid = lax.axis_index("s")

    zero16 = jnp.zeros((L,), jnp.float32)
    for r in range(32):
        for j in range(D // L):
            z_v[r, pl.ds(j * L, L)] = zero16
    rows_per_sub = NP // NS
    def _zero_body(i, carry):
        pltpu.sync_copy(z_v, acc_sh.at[pl.ds(sid * rows_per_sub + i * 32, 32)])
        return carry
    lax.fori_loop(0, rows_per_sub // 32, _zero_body, 0)
    plsc.subcore_barrier()

    def _body(i, carry):
        rel = sid + NS * i
        @pl.when(rel < CHUNKS_PER_CORE)
        def _():
            k = cid * CHUNKS_PER_CORE + rel
            pltpu.sync_copy(src_hbm.at[k], idx_v.at[0])
            pltpu.sync_copy(dst_hbm.at[k], idx_v.at[1])
            pltpu.sync_copy(xws_hbm.at[idx_v.at[0]], rows_v)
            pltpu.sync_copy(rows_v, acc_sh.at[idx_v.at[1]], add=True)
        return carry
    lax.fori_loop(0, ITERS, _body, 0)
    plsc.subcore_barrier()

    pltpu.sync_copy(
        acc_sh.at[pl.ds(sid * rows_per_sub, rows_per_sub)],
        out_hbm.at[cid, pl.ds(sid * rows_per_sub, rows_per_sub)],
    )


_agg_kernel = functools.partial(
    pl.kernel,
    out_type=jax.ShapeDtypeStruct((NC, NP, D), jnp.float32),
    mesh=_mesh,
    scratch_types=[
        pltpu.VMEM((2, CHUNK), jnp.int32),
        pltpu.VMEM((CHUNK, D), jnp.float32),
        pltpu.VMEM((32, D), jnp.float32),
        pltpu.VMEM_SHARED((NP, D), jnp.float32),
    ],
)(_agg_kernel_body)


def _mm_body(x_ref, w_ref, deg_ref, xws_ref):
    dis = lax.rsqrt(deg_ref[...])[:, 0]
    xw = jax.lax.dot_general(
        x_ref[...], w_ref[...], (((1,), (0,)), ((), ())),
        preferred_element_type=jnp.float32,
        precision=jax.lax.Precision.HIGHEST,
    )
    xws_ref[...] = xw * dis[:, None]


BM = 2000


def _mm(x, W, deg):
    return pl.pallas_call(
        _mm_body,
        grid=(N // BM,),
        in_specs=[
            pl.BlockSpec((BM, D), lambda i: (i, 0)),
            pl.BlockSpec((D, D), lambda i: (0, 0)),
            pl.BlockSpec((BM, 1), lambda i: (i, 0)),
        ],
        out_specs=pl.BlockSpec((BM, D), lambda i: (i, 0)),
        out_shape=jax.ShapeDtypeStruct((N, D), jnp.float32),
    )(x, W, deg)


def _comb_body(p_ref, xws_ref, deg_ref, b_ref, o_ref):
    dis = lax.rsqrt(deg_ref[...])[:, 0]
    p = p_ref[...]
    o_ref[...] = dis[:, None] * (p[0] + p[1] + xws_ref[...]) + b_ref[...][None, :]


def _combine(p, xws, deg, b):
    return pl.pallas_call(
        _comb_body,
        grid=(N // BM,),
        in_specs=[
            pl.BlockSpec((NC, BM, D), lambda i: (0, i, 0)),
            pl.BlockSpec((BM, D), lambda i: (i, 0)),
            pl.BlockSpec((BM, 1), lambda i: (i, 0)),
            pl.BlockSpec((D,), lambda i: (0,)),
        ],
        out_specs=pl.BlockSpec((BM, D), lambda i: (i, 0)),
        out_shape=jax.ShapeDtypeStruct((N, D), jnp.float32),
    )(p, xws, deg, b)


def kernel(x, edge_index, W, b):
    src2d = edge_index[0].reshape(NCHUNKS, CHUNK)
    dst2d = edge_index[1].reshape(NCHUNKS, CHUNK)
    dp = _deg_kernel(dst2d)
    deg = (dp[0, :N, 0] + dp[1, :N, 0] + 1.0)[:, None]
    xws = _mm(x, W, deg)
    p = _agg_kernel(xws, src2d, dst2d)
    return _combine(p, xws, deg, b)


# speedup vs baseline: 36.5659x; 1.7069x over previous
import functools

import jax
import jax.numpy as jnp
from jax import lax
from jax.experimental import pallas as pl
from jax.experimental.pallas import tpu as pltpu
from jax.experimental.pallas import tpu_sc as plsc

N = 10000
E = 320000
D = 128

CHUNK = 128
NCHUNKS = E // CHUNK
NC, NS, L = 2, 16, 16
NW = NC * NS
NP = 10240
CHUNKS_PER_CORE = NCHUNKS // NC
ITERS = -(-CHUNKS_PER_CORE // NS)

_mesh = plsc.VectorSubcoreMesh(core_axis_name="c", subcore_axis_name="s")


NB = 4
ROWS_PER_SUB = NP // NS


def _deg_kernel_body(dst_hbm, out_hbm, idx_v, ones_v, isem, ssem, hist_sh):
    cid = lax.axis_index("c")
    sid = lax.axis_index("s")
    base = cid * CHUNKS_PER_CORE

    one16 = jnp.ones((L,), jnp.float32)
    zero16 = jnp.zeros((L,), jnp.float32)
    for r in range(32):
        ones_v[r, :] = zero16
    def _zero_body(i, carry):
        pltpu.sync_copy(
            ones_v.at[pl.ds(0, 32)],
            hist_sh.at[pl.ds(sid * ROWS_PER_SUB + i * 32, 32)],
        )
        return carry
    lax.fori_loop(0, ROWS_PER_SUB // 32, _zero_body, 0)
    for r in range(CHUNK):
        ones_v[r, :] = one16
    plsc.subcore_barrier()

    def valid(j):
        return sid + NS * j < CHUNKS_PER_CORE

    def chunk(j):
        return base + sid + NS * j

    def idx_pair(j):
        b = j % NB
        return dst_hbm.at[chunk(j)], idx_v.at[b]

    def start_idx(j):
        if 0 <= j < ITERS:
            @pl.when(valid(j))
            def _():
                s, d = idx_pair(j)
                pltpu.async_copy(s, d, isem.at[j % NB])

    def wait_idx(j):
        @pl.when(valid(j))
        def _():
            s, d = idx_pair(j)
            pltpu.make_async_copy(s, d, isem.at[j % NB]).wait()

    def sct_pair(j):
        return ones_v, hist_sh.at[idx_v.at[j % NB]]

    def start_sct(j):
        @pl.when(valid(j))
        def _():
            s, d = sct_pair(j)
            pltpu.async_copy(s, d, ssem.at[j % NB], add=True)

    def wait_sct(j):
        if 0 <= j < ITERS:
            @pl.when(valid(j))
            def _():
                s, d = sct_pair(j)
                pltpu.make_async_copy(s, d, ssem.at[j % NB]).wait()

    for j in range(NB):
        start_idx(j)
    for i in range(ITERS):
        wait_idx(i)
        start_sct(i)
        if i >= 1:
            wait_sct(i - 1)
            start_idx(i + NB - 1)
    wait_sct(ITERS - 1)
    plsc.subcore_barrier()

    pltpu.sync_copy(
        hist_sh.at[pl.ds(sid * ROWS_PER_SUB, ROWS_PER_SUB)],
        out_hbm.at[cid, pl.ds(sid * ROWS_PER_SUB, ROWS_PER_SUB)],
    )


_deg_kernel = functools.partial(
    pl.kernel,
    out_type=jax.ShapeDtypeStruct((NC, NP, L), jnp.float32),
    mesh=_mesh,
    scratch_types=[
        pltpu.VMEM((NB, CHUNK), jnp.int32),
        pltpu.VMEM((CHUNK, L), jnp.float32),
        pltpu.SemaphoreType.DMA((NB,)),
        pltpu.SemaphoreType.DMA((NB,)),
        pltpu.VMEM_SHARED((NP, L), jnp.float32),
    ],
)(_deg_kernel_body)


NBR = 2
NBI = 8


def _agg_kernel_body(xws_hbm, ei_hbm, out_hbm, ei_v, rows_v,
                     isem, gsem, ssem, acc_sh):
    cid = lax.axis_index("c")
    sid = lax.axis_index("s")
    base = cid * CHUNKS_PER_CORE

    zero16 = jnp.zeros((L,), jnp.float32)
    for r in range(32):
        for j in range(D // L):
            rows_v[0, r, pl.ds(j * L, L)] = zero16
    def _zero_body(i, carry):
        pltpu.sync_copy(rows_v.at[0, pl.ds(0, 32)],
                        acc_sh.at[pl.ds(sid * ROWS_PER_SUB + i * 32, 32)])
        return carry
    lax.fori_loop(0, ROWS_PER_SUB // 32, _zero_body, 0)
    plsc.subcore_barrier()

    def valid(j):
        return sid + NS * j < CHUNKS_PER_CORE

    def chunk(j):
        return base + sid + NS * j

    def idx_pair(j):
        return ei_hbm.at[:, chunk(j)], ei_v.at[j % NBI]

    def start_idx(j):
        if 0 <= j < ITERS:
            @pl.when(valid(j))
            def _():
                s, d = idx_pair(j)
                pltpu.async_copy(s, d, isem.at[j % NBI])

    def wait_idx(j):
        @pl.when(valid(j))
        def _():
            s, d = idx_pair(j)
            pltpu.make_async_copy(s, d, isem.at[j % NBI]).wait()

    def gth_pair(j):
        return xws_hbm.at[ei_v.at[j % NBI, 0]], rows_v.at[j % NBR]

    def start_gth(j):
        @pl.when(valid(j))
        def _():
            s, d = gth_pair(j)
            pltpu.async_copy(s, d, gsem.at[j % NBR])

    def wait_gth(j):
        @pl.when(valid(j))
        def _():
            s, d = gth_pair(j)
            pltpu.make_async_copy(s, d, gsem.at[j % NBR]).wait()

    def sct_pair(j):
        return rows_v.at[j % NBR], acc_sh.at[ei_v.at[j % NBI, 1]]

    def start_sct(j):
        @pl.when(valid(j))
        def _():
            s, d = sct_pair(j)
            pltpu.async_copy(s, d, ssem.at[j % NBR], add=True)

    def wait_sct(j):
        if 0 <= j < ITERS:
            @pl.when(valid(j))
            def _():
                s, d = sct_pair(j)
                pltpu.make_async_copy(s, d, ssem.at[j % NBR]).wait()

    for j in range(NBI):
        start_idx(j)
    for i in range(ITERS):
        if i >= 1:
            wait_gth(i - 1)
            start_sct(i - 1)
        if i >= 2:
            wait_sct(i - 2)
            start_idx(i - 2 + NBI)
        wait_idx(i)
        start_gth(i)
    wait_gth(ITERS - 1)
    start_sct(ITERS - 1)
    wait_sct(ITERS - 2)
    wait_sct(ITERS - 1)
    plsc.subcore_barrier()

    pltpu.sync_copy(
        acc_sh.at[pl.ds(sid * ROWS_PER_SUB, ROWS_PER_SUB)],
        out_hbm.at[cid, pl.ds(sid * ROWS_PER_SUB, ROWS_PER_SUB)],
    )


_agg_kernel = functools.partial(
    pl.kernel,
    out_type=jax.ShapeDtypeStruct((NC, NP, D), jnp.float32),
    mesh=_mesh,
    scratch_types=[
        pltpu.VMEM((NBI, 2, CHUNK), jnp.int32),
        pltpu.VMEM((NBR, CHUNK, D), jnp.float32),
        pltpu.SemaphoreType.DMA((NBI,)),
        pltpu.SemaphoreType.DMA((NBR,)),
        pltpu.SemaphoreType.DMA((NBR,)),
        pltpu.VMEM_SHARED((NP, D), jnp.float32),
    ],
)(_agg_kernel_body)


def _mm_body(x_ref, w_ref, deg_ref, xws_ref):
    dis = lax.rsqrt(deg_ref[...])[:, 0]
    xw = jax.lax.dot_general(
        x_ref[...], w_ref[...], (((1,), (0,)), ((), ())),
        preferred_element_type=jnp.float32,
        precision=jax.lax.Precision.HIGHEST,
    )
    xws_ref[...] = xw * dis[:, None]


BM = 2000


def _mm(x, W, deg):
    return pl.pallas_call(
        _mm_body,
        grid=(N // BM,),
        in_specs=[
            pl.BlockSpec((BM, D), lambda i: (i, 0)),
            pl.BlockSpec((D, D), lambda i: (0, 0)),
            pl.BlockSpec((BM, 1), lambda i: (i, 0)),
        ],
        out_specs=pl.BlockSpec((BM, D), lambda i: (i, 0)),
        out_shape=jax.ShapeDtypeStruct((N, D), jnp.float32),
    )(x, W, deg)


def _comb_body(p_ref, xws_ref, deg_ref, b_ref, o_ref):
    dis = lax.rsqrt(deg_ref[...])[:, 0]
    p = p_ref[...]
    o_ref[...] = dis[:, None] * (p[0] + p[1] + xws_ref[...]) + b_ref[...][None, :]


def _combine(p, xws, deg, b):
    return pl.pallas_call(
        _comb_body,
        grid=(N // BM,),
        in_specs=[
            pl.BlockSpec((NC, BM, D), lambda i: (0, i, 0)),
            pl.BlockSpec((BM, D), lambda i: (i, 0)),
            pl.BlockSpec((BM, 1), lambda i: (i, 0)),
            pl.BlockSpec((D,), lambda i: (0,)),
        ],
        out_specs=pl.BlockSpec((BM, D), lambda i: (i, 0)),
        out_shape=jax.ShapeDtypeStruct((N, D), jnp.float32),
    )(p, xws, deg, b)


def kernel(x, edge_index, W, b):
    ei3d = edge_index.reshape(2, NCHUNKS, CHUNK)
    dp = _deg_kernel(edge_index[1].reshape(NCHUNKS, CHUNK))
    deg = (dp[0, :N, 0] + dp[1, :N, 0] + 1.0)[:, None]
    xws = _mm(x, W, deg)
    p = _agg_kernel(xws, ei3d)
    return _combine(p, xws, deg, b)


# speedup vs baseline: 40.7743x; 1.1151x over previous
import functools

import jax
import jax.numpy as jnp
from jax import lax
from jax.experimental import pallas as pl
from jax.experimental.pallas import tpu as pltpu
from jax.experimental.pallas import tpu_sc as plsc

N = 10000
E = 320000
D = 128

CHUNK = 100
NCHUNKS = E // CHUNK
NC, NS, L = 2, 16, 16
NP = 10240
CHUNKS_PER_CORE = NCHUNKS // NC
ITERS = CHUNKS_PER_CORE // NS
ROWS_PER_SUB = NP // NS

_mesh = plsc.VectorSubcoreMesh(core_axis_name="c", subcore_axis_name="s")


NBD = 4


def _deg_kernel_body(dst_hbm, out_hbm, idx_v, ones_v, isem, ssem, hist_sh):
    cid = lax.axis_index("c")
    sid = lax.axis_index("s")
    base = cid * CHUNKS_PER_CORE

    one16 = jnp.ones((L,), jnp.float32)
    zero16 = jnp.zeros((L,), jnp.float32)
    for r in range(32):
        ones_v[r, :] = zero16
    def _zero_body(i, carry):
        pltpu.sync_copy(
            ones_v.at[pl.ds(0, 32)],
            hist_sh.at[pl.ds(sid * ROWS_PER_SUB + i * 32, 32)],
        )
        return carry
    lax.fori_loop(0, ROWS_PER_SUB // 32, _zero_body, 0)
    for r in range(CHUNK):
        ones_v[r, :] = one16
    plsc.subcore_barrier()

    def chunk(j):
        return base + sid + NS * j

    def idx_pair(j):
        return dst_hbm.at[chunk(j)], idx_v.at[j % NBD]

    def start_idx(j):
        if 0 <= j < ITERS:
            s, d = idx_pair(j)
            pltpu.async_copy(s, d, isem.at[j % NBD])

    def wait_idx(j):
        s, d = idx_pair(j)
        pltpu.make_async_copy(s, d, isem.at[j % NBD]).wait()

    def sct_pair(j):
        return ones_v, hist_sh.at[idx_v.at[j % NBD]]

    def start_sct(j):
        s, d = sct_pair(j)
        pltpu.async_copy(s, d, ssem.at[j % NBD], add=True)

    def wait_sct(j):
        if 0 <= j < ITERS:
            s, d = sct_pair(j)
            pltpu.make_async_copy(s, d, ssem.at[j % NBD]).wait()

    for j in range(NBD):
        start_idx(j)
    for i in range(ITERS):
        wait_idx(i)
        start_sct(i)
        if i >= 1:
            wait_sct(i - 1)
            start_idx(i + NBD - 1)
    wait_sct(ITERS - 1)
    plsc.subcore_barrier()

    pltpu.sync_copy(
        hist_sh.at[pl.ds(sid * ROWS_PER_SUB, ROWS_PER_SUB)],
        out_hbm.at[cid, pl.ds(sid * ROWS_PER_SUB, ROWS_PER_SUB)],
    )


_deg_kernel = functools.partial(
    pl.kernel,
    out_type=jax.ShapeDtypeStruct((NC, NP, L), jnp.float32),
    mesh=_mesh,
    scratch_types=[
        pltpu.VMEM((NBD, CHUNK), jnp.int32),
        pltpu.VMEM((CHUNK, L), jnp.float32),
        pltpu.SemaphoreType.DMA((NBD,)),
        pltpu.SemaphoreType.DMA((NBD,)),
        pltpu.VMEM_SHARED((NP, L), jnp.float32),
    ],
)(_deg_kernel_body)


NBR = 3
NBI = 8


def _agg_kernel_body(xws_hbm, ei_hbm, out_hbm, ei_v, rows_v,
                     isem, gsem, ssem, acc_sh):
    cid = lax.axis_index("c")
    sid = lax.axis_index("s")
    base = cid * CHUNKS_PER_CORE

    zero16 = jnp.zeros((L,), jnp.float32)
    for r in range(32):
        for j in range(D // L):
            rows_v[0, r, pl.ds(j * L, L)] = zero16
    def _zero_body(i, carry):
        pltpu.sync_copy(rows_v.at[0, pl.ds(0, 32)],
                        acc_sh.at[pl.ds(sid * ROWS_PER_SUB + i * 32, 32)])
        return carry
    lax.fori_loop(0, ROWS_PER_SUB // 32, _zero_body, 0)
    plsc.subcore_barrier()

    def chunk(j):
        return base + sid + NS * j

    def idx_pair(j):
        return ei_hbm.at[:, chunk(j)], ei_v.at[j % NBI]

    def start_idx(j):
        if 0 <= j < ITERS:
            s, d = idx_pair(j)
            pltpu.async_copy(s, d, isem.at[j % NBI])

    def wait_idx(j):
        s, d = idx_pair(j)
        pltpu.make_async_copy(s, d, isem.at[j % NBI]).wait()

    def gth_pair(j):
        return xws_hbm.at[ei_v.at[j % NBI, 0]], rows_v.at[j % NBR]

    def start_gth(j):
        s, d = gth_pair(j)
        pltpu.async_copy(s, d, gsem.at[j % NBR])

    def wait_gth(j):
        s, d = gth_pair(j)
        pltpu.make_async_copy(s, d, gsem.at[j % NBR]).wait()

    def sct_pair(j):
        return rows_v.at[j % NBR], acc_sh.at[ei_v.at[j % NBI, 1]]

    def start_sct(j):
        s, d = sct_pair(j)
        pltpu.async_copy(s, d, ssem.at[j % NBR], add=True)

    def wait_sct(j):
        if 0 <= j < ITERS:
            s, d = sct_pair(j)
            pltpu.make_async_copy(s, d, ssem.at[j % NBR]).wait()

    for j in range(NBI):
        start_idx(j)
    for i in range(ITERS):
        if i >= NBR:
            wait_sct(i - NBR)
            start_idx(i - NBR + NBI)
        wait_idx(i)
        start_gth(i)
        if i >= 1:
            wait_gth(i - 1)
            start_sct(i - 1)
    wait_gth(ITERS - 1)
    start_sct(ITERS - 1)
    for j in range(NBR, 0, -1):
        wait_sct(ITERS - j)
    plsc.subcore_barrier()

    pltpu.sync_copy(
        acc_sh.at[pl.ds(sid * ROWS_PER_SUB, ROWS_PER_SUB)],
        out_hbm.at[cid, pl.ds(sid * ROWS_PER_SUB, ROWS_PER_SUB)],
    )


_agg_kernel = functools.partial(
    pl.kernel,
    out_type=jax.ShapeDtypeStruct((NC, NP, D), jnp.float32),
    mesh=_mesh,
    scratch_types=[
        pltpu.VMEM((NBI, 2, CHUNK), jnp.int32),
        pltpu.VMEM((NBR, CHUNK, D), jnp.float32),
        pltpu.SemaphoreType.DMA((NBI,)),
        pltpu.SemaphoreType.DMA((NBR,)),
        pltpu.SemaphoreType.DMA((NBR,)),
        pltpu.VMEM_SHARED((NP, D), jnp.float32),
    ],
)(_agg_kernel_body)


def _mm_body(x_ref, w_ref, deg_ref, xws_ref):
    dis = lax.rsqrt(deg_ref[...])[:, 0]
    xw = jax.lax.dot_general(
        x_ref[...], w_ref[...], (((1,), (0,)), ((), ())),
        preferred_element_type=jnp.float32,
        precision=jax.lax.Precision.HIGHEST,
    )
    xws_ref[...] = xw * dis[:, None]


BM = 2000


def _mm(x, W, deg):
    return pl.pallas_call(
        _mm_body,
        grid=(N // BM,),
        in_specs=[
            pl.BlockSpec((BM, D), lambda i: (i, 0)),
            pl.BlockSpec((D, D), lambda i: (0, 0)),
            pl.BlockSpec((BM, 1), lambda i: (i, 0)),
        ],
        out_specs=pl.BlockSpec((BM, D), lambda i: (i, 0)),
        out_shape=jax.ShapeDtypeStruct((N, D), jnp.float32),
    )(x, W, deg)


def _comb_body(p_ref, xws_ref, deg_ref, b_ref, o_ref):
    dis = lax.rsqrt(deg_ref[...])[:, 0]
    p = p_ref[...]
    o_ref[...] = dis[:, None] * (p[0] + p[1] + xws_ref[...]) + b_ref[...][None, :]


def _combine(p, xws, deg, b):
    return pl.pallas_call(
        _comb_body,
        grid=(N // BM,),
        in_specs=[
            pl.BlockSpec((NC, BM, D), lambda i: (0, i, 0)),
            pl.BlockSpec((BM, D), lambda i: (i, 0)),
            pl.BlockSpec((BM, 1), lambda i: (i, 0)),
            pl.BlockSpec((D,), lambda i: (0,)),
        ],
        out_specs=pl.BlockSpec((BM, D), lambda i: (i, 0)),
        out_shape=jax.ShapeDtypeStruct((N, D), jnp.float32),
    )(p, xws, deg, b)


def kernel(x, edge_index, W, b):
    ei3d = edge_index.reshape(2, NCHUNKS, CHUNK)
    dp = _deg_kernel(edge_index[1].reshape(NCHUNKS, CHUNK))
    deg = (dp[0, :N, 0] + dp[1, :N, 0] + 1.0)[:, None]
    xws = _mm(x, W, deg)
    p = _agg_kernel(xws, ei3d)
    return _combine(p, xws, deg, b)


# speedup vs baseline: 43.6202x; 1.0698x over previous
import functools

import jax
import jax.numpy as jnp
from jax import lax
from jax.experimental import pallas as pl
from jax.experimental.pallas import tpu as pltpu
from jax.experimental.pallas import tpu_sc as plsc

N = 10000
E = 320000
D = 128

CHUNK = 80
NCHUNKS = E // CHUNK
NC, NS, L = 2, 16, 16
NP = 10240
CHUNKS_PER_CORE = NCHUNKS // NC
ITERS = CHUNKS_PER_CORE // NS
ROWS_PER_SUB = NP // NS

_mesh = plsc.VectorSubcoreMesh(core_axis_name="c", subcore_axis_name="s")


NBD = 6


def _deg_kernel_body(dst_hbm, out_hbm, idx_v, ones_v, isem, ssem, hist_sh):
    cid = lax.axis_index("c")
    sid = lax.axis_index("s")
    base = cid * CHUNKS_PER_CORE

    one16 = jnp.ones((L,), jnp.float32)
    zero16 = jnp.zeros((L,), jnp.float32)
    for r in range(32):
        ones_v[r, :] = zero16
    def _zero_body(i, carry):
        pltpu.sync_copy(
            ones_v.at[pl.ds(0, 32)],
            hist_sh.at[pl.ds(sid * ROWS_PER_SUB + i * 32, 32)],
        )
        return carry
    lax.fori_loop(0, ROWS_PER_SUB // 32, _zero_body, 0)
    for r in range(CHUNK):
        ones_v[r, :] = one16
    plsc.subcore_barrier()

    def chunk(j):
        return base + sid + NS * j

    def idx_pair(j):
        return dst_hbm.at[chunk(j)], idx_v.at[j % NBD]

    def start_idx(j):
        if 0 <= j < ITERS:
            s, d = idx_pair(j)
            pltpu.async_copy(s, d, isem.at[j % NBD])

    def wait_idx(j):
        s, d = idx_pair(j)
        pltpu.make_async_copy(s, d, isem.at[j % NBD]).wait()

    def sct_pair(j):
        return ones_v, hist_sh.at[idx_v.at[j % NBD]]

    def start_sct(j):
        s, d = sct_pair(j)
        pltpu.async_copy(s, d, ssem.at[j % NBD], add=True)

    def wait_sct(j):
        if 0 <= j < ITERS:
            s, d = sct_pair(j)
            pltpu.make_async_copy(s, d, ssem.at[j % NBD]).wait()

    for j in range(NBD):
        start_idx(j)
    for i in range(ITERS):
        wait_idx(i)
        start_sct(i)
        if i >= 1:
            wait_sct(i - 1)
            start_idx(i + NBD - 1)
    wait_sct(ITERS - 1)
    plsc.subcore_barrier()

    pltpu.sync_copy(
        hist_sh.at[pl.ds(sid * ROWS_PER_SUB, ROWS_PER_SUB)],
        out_hbm.at[cid, pl.ds(sid * ROWS_PER_SUB, ROWS_PER_SUB)],
    )


_deg_kernel = functools.partial(
    pl.kernel,
    out_type=jax.ShapeDtypeStruct((NC, NP, L), jnp.float32),
    mesh=_mesh,
    scratch_types=[
        pltpu.VMEM((NBD, CHUNK), jnp.int32),
        pltpu.VMEM((CHUNK, L), jnp.float32),
        pltpu.SemaphoreType.DMA((NBD,)),
        pltpu.SemaphoreType.DMA((NBD,)),
        pltpu.VMEM_SHARED((NP, L), jnp.float32),
    ],
)(_deg_kernel_body)


NBR = 4
NBI = 8


def _agg_kernel_body(xws_hbm, ei_hbm, out_hbm, ei_v, rows_v,
                     isem, gsem, ssem, acc_sh):
    cid = lax.axis_index("c")
    sid = lax.axis_index("s")
    base = cid * CHUNKS_PER_CORE

    zero16 = jnp.zeros((L,), jnp.float32)
    for r in range(32):
        for j in range(D // L):
            rows_v[0, r, pl.ds(j * L, L)] = zero16
    def _zero_body(i, carry):
        pltpu.sync_copy(rows_v.at[0, pl.ds(0, 32)],
                        acc_sh.at[pl.ds(sid * ROWS_PER_SUB + i * 32, 32)])
        return carry
    lax.fori_loop(0, ROWS_PER_SUB // 32, _zero_body, 0)
    plsc.subcore_barrier()

    def chunk(j):
        return base + sid + NS * j

    def idx_pair(j):
        return ei_hbm.at[:, chunk(j)], ei_v.at[j % NBI]

    def start_idx(j):
        if 0 <= j < ITERS:
            s, d = idx_pair(j)
            pltpu.async_copy(s, d, isem.at[j % NBI])

    def wait_idx(j):
        s, d = idx_pair(j)
        pltpu.make_async_copy(s, d, isem.at[j % NBI]).wait()

    def gth_pair(j):
        return xws_hbm.at[ei_v.at[j % NBI, 0]], rows_v.at[j % NBR]

    def start_gth(j):
        s, d = gth_pair(j)
        pltpu.async_copy(s, d, gsem.at[j % NBR])

    def wait_gth(j):
        s, d = gth_pair(j)
        pltpu.make_async_copy(s, d, gsem.at[j % NBR]).wait()

    def sct_pair(j):
        return rows_v.at[j % NBR], acc_sh.at[ei_v.at[j % NBI, 1]]

    def start_sct(j):
        s, d = sct_pair(j)
        pltpu.async_copy(s, d, ssem.at[j % NBR], add=True)

    def wait_sct(j):
        if 0 <= j < ITERS:
            s, d = sct_pair(j)
            pltpu.make_async_copy(s, d, ssem.at[j % NBR]).wait()

    for j in range(NBI):
        start_idx(j)
    for i in range(ITERS):
        if i >= NBR:
            wait_sct(i - NBR)
            start_idx(i - NBR + NBI)
        wait_idx(i)
        start_gth(i)
        if i >= NBR - 1:
            wait_gth(i - (NBR - 1))
            start_sct(i - (NBR - 1))
    for k in range(NBR - 1, 0, -1):
        wait_gth(ITERS - k)
        start_sct(ITERS - k)
    for k in range(NBR, 0, -1):
        wait_sct(ITERS - k)
    plsc.subcore_barrier()

    pltpu.sync_copy(
        acc_sh.at[pl.ds(sid * ROWS_PER_SUB, ROWS_PER_SUB)],
        out_hbm.at[cid, pl.ds(sid * ROWS_PER_SUB, ROWS_PER_SUB)],
    )


_agg_kernel = functools.partial(
    pl.kernel,
    out_type=jax.ShapeDtypeStruct((NC, NP, D), jnp.float32),
    mesh=_mesh,
    scratch_types=[
        pltpu.VMEM((NBI, 2, CHUNK), jnp.int32),
        pltpu.VMEM((NBR, CHUNK, D), jnp.float32),
        pltpu.SemaphoreType.DMA((NBI,)),
        pltpu.SemaphoreType.DMA((NBR,)),
        pltpu.SemaphoreType.DMA((NBR,)),
        pltpu.VMEM_SHARED((NP, D), jnp.float32),
    ],
)(_agg_kernel_body)


def _mm_body(x_ref, w_ref, deg_ref, xws_ref):
    dis = lax.rsqrt(deg_ref[...])[:, 0]
    xw = jax.lax.dot_general(
        x_ref[...], w_ref[...], (((1,), (0,)), ((), ())),
        preferred_element_type=jnp.float32,
        precision=jax.lax.Precision.HIGHEST,
    )
    xws_ref[...] = xw * dis[:, None]


BM = 2000


def _mm(x, W, deg):
    return pl.pallas_call(
        _mm_body,
        grid=(N // BM,),
        in_specs=[
            pl.BlockSpec((BM, D), lambda i: (i, 0)),
            pl.BlockSpec((D, D), lambda i: (0, 0)),
            pl.BlockSpec((BM, 1), lambda i: (i, 0)),
        ],
        out_specs=pl.BlockSpec((BM, D), lambda i: (i, 0)),
        out_shape=jax.ShapeDtypeStruct((N, D), jnp.float32),
    )(x, W, deg)


def _comb_body(p_ref, xws_ref, deg_ref, b_ref, o_ref):
    dis = lax.rsqrt(deg_ref[...])[:, 0]
    p = p_ref[...]
    o_ref[...] = dis[:, None] * (p[0] + p[1] + xws_ref[...]) + b_ref[...][None, :]


def _combine(p, xws, deg, b):
    return pl.pallas_call(
        _comb_body,
        grid=(N // BM,),
        in_specs=[
            pl.BlockSpec((NC, BM, D), lambda i: (0, i, 0)),
            pl.BlockSpec((BM, D), lambda i: (i, 0)),
            pl.BlockSpec((BM, 1), lambda i: (i, 0)),
            pl.BlockSpec((D,), lambda i: (0,)),
        ],
        out_specs=pl.BlockSpec((BM, D), lambda i: (i, 0)),
        out_shape=jax.ShapeDtypeStruct((N, D), jnp.float32),
    )(p, xws, deg, b)


def kernel(x, edge_index, W, b):
    ei3d = edge_index.reshape(2, NCHUNKS, CHUNK)
    dp = _deg_kernel(edge_index[1].reshape(NCHUNKS, CHUNK))
    deg = (dp[0, :N, 0] + dp[1, :N, 0] + 1.0)[:, None]
    xws = _mm(x, W, deg)
    p = _agg_kernel(xws, ei3d)
    return _combine(p, xws, deg, b)
